# baseline (speedup 1.0000x reference)
import functools

import jax
import jax.numpy as jnp
from jax import lax
from jax.experimental import pallas as pl
from jax.experimental.pallas import tpu as pltpu
from jax.experimental.pallas import tpu_sc as plsc

_N_RAYS = 8192
_N_SAMPLE = 256
_N_GRID = 128
_NEAR = 0.2
_ROW = _N_SAMPLE * 7
_GSZ = 16

_info = plsc.get_sparse_core_info()
_NC, _NS, _L = _info.num_cores, _info.num_subcores, _info.num_lanes
_NW = _NC * _NS
_RAYS_PER_W = _N_RAYS // _NW
_GROUPS = _RAYS_PER_W // _GSZ
_NWORDS = _N_GRID * _N_GRID * _N_GRID // 32

_mesh = plsc.VectorSubcoreMesh(core_axis_name="c", subcore_axis_name="s")


@functools.partial(
    pl.kernel,
    mesh=_mesh,
    out_type=[
        jax.ShapeDtypeStruct((_N_RAYS * _ROW,), jnp.float32),
        jax.ShapeDtypeStruct((_N_RAYS,), jnp.int32),
    ],
    scratch_types=[
        pltpu.VMEM((_NWORDS,), jnp.int32),
        pltpu.VMEM((6, _RAYS_PER_W), jnp.float32),
        pltpu.VMEM((16,), jnp.float32),
        pltpu.VMEM((2 * _GSZ * _ROW,), jnp.float32),
        pltpu.VMEM((_RAYS_PER_W,), jnp.int32),
        pltpu.SemaphoreType.DMA,
    ],
    compiler_params=pltpu.CompilerParams(needs_layout_passes=False),
)
def _march_kernel(rays_hbm, words_hbm, params_hbm,
                  coords_hbm, counts_hbm,
                  bits_v, rays_v, par_v, stag_v, counts_v, dma_sem):
    wid = lax.axis_index("s") * _NC + lax.axis_index("c")
    base = wid * _RAYS_PER_W

    pltpu.sync_copy(words_hbm, bits_v)
    pltpu.sync_copy(rays_hbm.at[:, pl.ds(base, _RAYS_PER_W)], rays_v)
    pltpu.sync_copy(params_hbm, par_v)

    pv = par_v[pl.ds(0, 16)]
    a0x, a0y, a0z = pv[0], pv[1], pv[2]
    a1x, a1y, a1z = pv[3], pv[4], pv[5]
    ms = pv[6]
    sx, sy, sz = a1x - a0x, a1y - a0y, a1z - a0z

    lanef = lax.iota(jnp.int32, _L).astype(jnp.float32)
    lanei = lax.iota(jnp.int32, _L)
    zv = jnp.zeros((_L,), jnp.float32)
    ngf = jnp.float32(_N_GRID)

    dtv = jnp.broadcast_to(ms, (_L,))
    gwords = _GSZ * _ROW

    def group_body(g, hws):
        hw0, hw1 = hws
        b = g & 1
        boff = b * gwords
        hwv = jnp.where(b == 0, hw0, hw1)
        @pl.when(g >= 2)
        def _():
            pltpu.make_async_copy(
                stag_v.at[pl.ds(boff, gwords)],
                coords_hbm.at[pl.ds((base + (g - 2) * _GSZ) * _ROW, gwords)],
                dma_sem).wait()
        gbase = g * _GSZ
        oxv = rays_v[0, pl.ds(gbase, _GSZ)]
        oyv = rays_v[1, pl.ds(gbase, _GSZ)]
        ozv = rays_v[2, pl.ds(gbase, _GSZ)]
        dxv = rays_v[3, pl.ds(gbase, _GSZ)]
        dyv = rays_v[4, pl.ds(gbase, _GSZ)]
        dzv = rays_v[5, pl.ds(gbase, _GSZ)]

        sdx = jnp.where(jnp.abs(dxv) > 1e-10, dxv, jnp.float32(1e-10))
        sdy = jnp.where(jnp.abs(dyv) > 1e-10, dyv, jnp.float32(1e-10))
        sdz = jnp.where(jnp.abs(dzv) > 1e-10, dzv, jnp.float32(1e-10))
        ivx, ivy, ivz = 1.0 / sdx, 1.0 / sdy, 1.0 / sdz
        t0x, t1x = (a0x - oxv) * ivx, (a1x - oxv) * ivx
        t0y, t1y = (a0y - oyv) * ivy, (a1y - oyv) * ivy
        t0z, t1z = (a0z - ozv) * ivz, (a1z - ozv) * ivz
        tnv = jnp.maximum(jnp.maximum(jnp.minimum(t0x, t1x),
                                      jnp.minimum(t0y, t1y)),
                          jnp.minimum(t0z, t1z))
        tfv = jnp.minimum(jnp.minimum(jnp.maximum(t0x, t1x),
                                      jnp.maximum(t0y, t1y)),
                          jnp.maximum(t0z, t1z))
        tnv = jnp.maximum(tnv, jnp.float32(_NEAR))
        qv = (tfv - tnv) / ms
        nchv = jnp.minimum(
            jnp.int32(_N_SAMPLE // _L),
            jnp.maximum(jnp.int32(0),
                        (qv * jnp.float32(1.0 / _L)).astype(jnp.int32) + 1))

        for rl in range(_GSZ):
            ox, oy, oz = oxv[rl], oyv[rl], ozv[rl]
            dx, dy, dz = dxv[rl], dyv[rl], dzv[rl]
            tn, tf, nch = tnv[rl], tfv[rl], nchv[rl]

            khi = lax.shift_right_logical(hwv[rl] * 7 + (_L - 1), 4)
            rowoff = boff + rl * _ROW

            def zero_body(k, _, rowoff=rowoff):
                stag_v[pl.ds(rowoff + k * _L, _L)] = zv
                return 0
            lax.fori_loop(0, khi, zero_body, 0)

            rowoff_splat = jnp.broadcast_to(rowoff, (_L,))
            dxs = jnp.broadcast_to(dx, (_L,))
            dys = jnp.broadcast_to(dy, (_L,))
            dzs = jnp.broadcast_to(dz, (_L,))

            def chunk_body(c, cnt, tn=tn, tf=tf, dx=dx, dy=dy, dz=dz,
                           ox=ox, oy=oy, oz=oz, rowoff_splat=rowoff_splat,
                           dtv=dtv, dxs=dxs, dys=dys, dzs=dzs):
                jv = lanef + (c * _L).astype(jnp.float32)
                tv = tn + (jv + 0.5) * ms
                px = ox + tv * dx
                py = oy + tv * dy
                pz = oz + tv * dz
                inb = ((px >= a0x) & (px <= a1x) &
                       (py >= a0y) & (py <= a1y) &
                       (pz >= a0z) & (pz <= a1z))
                seg = tv < tf
                gx = jnp.clip(((px - a0x) / sx * ngf).astype(jnp.int32),
                              0, _N_GRID - 1)
                gy = jnp.clip(((py - a0y) / sy * ngf).astype(jnp.int32),
                              0, _N_GRID - 1)
                gz = jnp.clip(((pz - a0z) / sz * ngf).astype(jnp.int32),
                              0, _N_GRID - 1)
                lin = (gx * _N_GRID + gy) * _N_GRID + gz
                wi = lax.shift_right_logical(lin, 5)
                bi = lin & 31
                w = plsc.load_gather(bits_v, [wi])
                occ = (lax.shift_right_logical(w, bi) & 1) == 1
                mask = inb & seg & occ
                mi = mask.astype(jnp.int32)
                cs = plsc.cumsum(mi)
                pos = (cnt + cs - 1) * 7 + rowoff_splat
                plsc.store_scatter(stag_v, [pos], px, mask=mask)
                plsc.store_scatter(stag_v, [pos + 1], py, mask=mask)
                plsc.store_scatter(stag_v, [pos + 2], pz, mask=mask)
                plsc.store_scatter(stag_v, [pos + 3], dtv, mask=mask)
                plsc.store_scatter(stag_v, [pos + 4], dxs, mask=mask)
                plsc.store_scatter(stag_v, [pos + 5], dys, mask=mask)
                plsc.store_scatter(stag_v, [pos + 6], dzs, mask=mask)
                return cnt + cs[_L - 1]

            cnt = lax.fori_loop(0, nch, chunk_body, jnp.int32(0))
            ridx = jnp.full((_L,), gbase + rl, jnp.int32)
            plsc.store_scatter(counts_v, [ridx],
                               jnp.broadcast_to(cnt, (_L,)), mask=lanei == 0)
            hwv = jnp.where(lanei == rl, jnp.broadcast_to(cnt, (_L,)), hwv)

        pltpu.async_copy(
            stag_v.at[pl.ds(boff, gwords)],
            coords_hbm.at[pl.ds((base + gbase) * _ROW, gwords)],
            dma_sem)
        hw0 = jnp.where(b == 0, hwv, hw0)
        hw1 = jnp.where(b == 0, hw1, hwv)
        return hw0, hw1

    hw_init = jnp.full((_L,), _N_SAMPLE, jnp.int32)
    lax.fori_loop(0, _GROUPS, group_body, (hw_init, hw_init))
    for gt in (_GROUPS - 2, _GROUPS - 1):
        pltpu.make_async_copy(
            stag_v.at[pl.ds((gt & 1) * gwords, gwords)],
            coords_hbm.at[pl.ds((base + gt * _GSZ) * _ROW, gwords)],
            dma_sem).wait()
    pltpu.sync_copy(counts_v, counts_hbm.at[pl.ds(base, _RAYS_PER_W)])


@functools.partial(
    pl.kernel,
    mesh=_mesh,
    out_type=jax.ShapeDtypeStruct((_N_RAYS,), jnp.int32),
    scratch_types=[
        pltpu.VMEM((_N_RAYS,), jnp.int32),
        pltpu.VMEM((_N_RAYS,), jnp.int32),
    ],
    compiler_params=pltpu.CompilerParams(needs_layout_passes=False),
)
def _offsets_kernel(counts_hbm, offs_hbm, cv, ov):
    wid = lax.axis_index("s") * _NC + lax.axis_index("c")

    @pl.when(wid == 0)
    def _():
        pltpu.sync_copy(counts_hbm, cv)

        def body(k, b):
            x = cv[pl.ds(k * _L, _L)]
            cs = plsc.cumsum(x)
            ov[pl.ds(k * _L, _L)] = (b + cs) - x
            return b + jnp.sum(x)

        lax.fori_loop(0, _N_RAYS // _L, body, jnp.int32(0))
        pltpu.sync_copy(ov, offs_hbm)


def kernel(rays_o, rays_d, bitfield, n_sample, min_step, max_step, aabb_range, n_grid, n_cascades):
    bits = bitfield.reshape(_NWORDS, 32).astype(jnp.uint32)
    words = jnp.sum(bits << jnp.arange(32, dtype=jnp.uint32)[None, :],
                    axis=1, dtype=jnp.uint32)
    words = lax.bitcast_convert_type(words, jnp.int32)

    rays = jnp.concatenate([rays_o.T, rays_d.T], axis=0)

    ms = jnp.asarray(min_step, jnp.float32)
    params = jnp.concatenate([
        aabb_range[0].astype(jnp.float32),
        aabb_range[1].astype(jnp.float32),
        ms[None],
        jnp.zeros((9,), jnp.float32),
    ])

    coords_flat, counts = _march_kernel(rays, words, params)
    offsets = _offsets_kernel(counts)
    coords = coords_flat.reshape(_N_RAYS, _N_SAMPLE, 7)
    numsteps_out = jnp.stack([counts, offsets], axis=-1)
    return coords, numsteps_out


# speedup vs baseline: 3.9195x; 3.9195x over previous
import functools

import jax
import jax.numpy as jnp
from jax import lax
from jax.experimental import pallas as pl
from jax.experimental.pallas import tpu as pltpu
from jax.experimental.pallas import tpu_sc as plsc

_N_RAYS = 8192
_N_SAMPLE = 256
_N_GRID = 128
_NEAR = 0.2
_ROW = _N_SAMPLE * 7
_GSZ = 16

_info = plsc.get_sparse_core_info()
_NC, _NS, _L = _info.num_cores, _info.num_subcores, _info.num_lanes
_NW = _NC * _NS
_RAYS_PER_W = _N_RAYS // _NW
_GROUPS = _RAYS_PER_W // _GSZ
_NWORDS = _N_GRID * _N_GRID * _N_GRID // 32

_mesh = plsc.VectorSubcoreMesh(core_axis_name="c", subcore_axis_name="s")


@functools.partial(
    pl.kernel,
    mesh=_mesh,
    out_type=[
        jax.ShapeDtypeStruct((_N_RAYS, _ROW), jnp.float32),
        jax.ShapeDtypeStruct((_N_RAYS,), jnp.int32),
    ],
    scratch_types=[
        pltpu.VMEM((_NWORDS,), jnp.int32),
        pltpu.VMEM((6, _RAYS_PER_W), jnp.float32),
        pltpu.VMEM((16,), jnp.float32),
        pltpu.VMEM((2, _GSZ, _ROW), jnp.float32),
        pltpu.VMEM((_RAYS_PER_W,), jnp.int32),
        pltpu.SemaphoreType.DMA,
    ],
    compiler_params=pltpu.CompilerParams(needs_layout_passes=False),
)
def _march_kernel(rays_hbm, words_hbm, params_hbm,
                  coords_hbm, counts_hbm,
                  bits_v, rays_v, par_v, stag_v, counts_v, dma_sem):
    wid = lax.axis_index("s") * _NC + lax.axis_index("c")
    base = wid * _RAYS_PER_W

    pltpu.sync_copy(words_hbm, bits_v)
    pltpu.sync_copy(rays_hbm.at[:, pl.ds(base, _RAYS_PER_W)], rays_v)
    pltpu.sync_copy(params_hbm, par_v)

    pv = par_v[pl.ds(0, 16)]
    a0x, a0y, a0z = pv[0], pv[1], pv[2]
    a1x, a1y, a1z = pv[3], pv[4], pv[5]
    ms = pv[6]
    sx, sy, sz = a1x - a0x, a1y - a0y, a1z - a0z

    lanef = lax.iota(jnp.int32, _L).astype(jnp.float32)
    lanei = lax.iota(jnp.int32, _L)
    zv = jnp.zeros((_L,), jnp.float32)
    ngf = jnp.float32(_N_GRID)

    dtv = jnp.broadcast_to(ms, (_L,))

    def pair_body(gg, hws):
      for b in range(2):
        hwv = hws[b]
        g = gg * 2 + b
        gbase = g * _GSZ

        @pl.when(gg >= 1)
        def _(b=b, g=g):
            pltpu.make_async_copy(
                stag_v.at[b],
                coords_hbm.at[pl.ds(base + (g - 2) * _GSZ, _GSZ)],
                dma_sem).wait()

        oxv = rays_v[0, pl.ds(gbase, _GSZ)]
        oyv = rays_v[1, pl.ds(gbase, _GSZ)]
        ozv = rays_v[2, pl.ds(gbase, _GSZ)]
        dxv = rays_v[3, pl.ds(gbase, _GSZ)]
        dyv = rays_v[4, pl.ds(gbase, _GSZ)]
        dzv = rays_v[5, pl.ds(gbase, _GSZ)]

        sdx = jnp.where(jnp.abs(dxv) > 1e-10, dxv, jnp.float32(1e-10))
        sdy = jnp.where(jnp.abs(dyv) > 1e-10, dyv, jnp.float32(1e-10))
        sdz = jnp.where(jnp.abs(dzv) > 1e-10, dzv, jnp.float32(1e-10))
        ivx, ivy, ivz = 1.0 / sdx, 1.0 / sdy, 1.0 / sdz
        t0x, t1x = (a0x - oxv) * ivx, (a1x - oxv) * ivx
        t0y, t1y = (a0y - oyv) * ivy, (a1y - oyv) * ivy
        t0z, t1z = (a0z - ozv) * ivz, (a1z - ozv) * ivz
        tnv = jnp.maximum(jnp.maximum(jnp.minimum(t0x, t1x),
                                      jnp.minimum(t0y, t1y)),
                          jnp.minimum(t0z, t1z))
        tfv = jnp.minimum(jnp.minimum(jnp.maximum(t0x, t1x),
                                      jnp.maximum(t0y, t1y)),
                          jnp.maximum(t0z, t1z))
        tnv = jnp.maximum(tnv, jnp.float32(_NEAR))
        qv = (tfv - tnv) / ms
        nchv = jnp.minimum(
            jnp.int32(_N_SAMPLE // _L),
            jnp.maximum(jnp.int32(0),
                        (qv * jnp.float32(1.0 / _L)).astype(jnp.int32) + 1))

        for rl in range(_GSZ):
            ox, oy, oz = oxv[rl], oyv[rl], ozv[rl]
            dx, dy, dz = dxv[rl], dyv[rl], dzv[rl]
            tn, tf, nch = tnv[rl], tfv[rl], nchv[rl]

            khi = lax.shift_right_logical(hwv[rl] * 7 + (_L - 1), 4)

            def zero_body(k, _, b=b, rl=rl):
                stag_v[b, rl, pl.ds(k * _L, _L)] = zv
                return 0
            lax.fori_loop(0, khi, zero_body, 0)

            b_splat = jnp.full((_L,), b, jnp.int32)
            rl_splat = jnp.full((_L,), rl, jnp.int32)
            dxs = jnp.broadcast_to(dx, (_L,))
            dys = jnp.broadcast_to(dy, (_L,))
            dzs = jnp.broadcast_to(dz, (_L,))

            def chunk_body(c, cnt, tn=tn, tf=tf, dx=dx, dy=dy, dz=dz,
                           ox=ox, oy=oy, oz=oz, rl_splat=rl_splat,
                           b_splat=b_splat, dtv=dtv, dxs=dxs, dys=dys,
                           dzs=dzs):
                jv = lanef + (c * _L).astype(jnp.float32)
                tv = tn + (jv + 0.5) * ms
                px = ox + tv * dx
                py = oy + tv * dy
                pz = oz + tv * dz
                inb = ((px >= a0x) & (px <= a1x) &
                       (py >= a0y) & (py <= a1y) &
                       (pz >= a0z) & (pz <= a1z))
                seg = tv < tf
                gx = jnp.clip(((px - a0x) / sx * ngf).astype(jnp.int32),
                              0, _N_GRID - 1)
                gy = jnp.clip(((py - a0y) / sy * ngf).astype(jnp.int32),
                              0, _N_GRID - 1)
                gz = jnp.clip(((pz - a0z) / sz * ngf).astype(jnp.int32),
                              0, _N_GRID - 1)
                lin = (gx * _N_GRID + gy) * _N_GRID + gz
                wi = lax.shift_right_logical(lin, 5)
                bi = lin & 31
                w = plsc.load_gather(bits_v, [wi])
                occ = (lax.shift_right_logical(w, bi) & 1) == 1
                mask = inb & seg & occ
                mi = mask.astype(jnp.int32)
                cs = plsc.cumsum(mi)
                pos = (cnt + cs - 1) * 7
                idx = [b_splat, rl_splat, pos]
                plsc.store_scatter(stag_v, idx, px, mask=mask)
                idx[2] = pos + 1
                plsc.store_scatter(stag_v, idx, py, mask=mask)
                idx[2] = pos + 2
                plsc.store_scatter(stag_v, idx, pz, mask=mask)
                idx[2] = pos + 3
                plsc.store_scatter(stag_v, idx, dtv, mask=mask)
                idx[2] = pos + 4
                plsc.store_scatter(stag_v, idx, dxs, mask=mask)
                idx[2] = pos + 5
                plsc.store_scatter(stag_v, idx, dys, mask=mask)
                idx[2] = pos + 6
                plsc.store_scatter(stag_v, idx, dzs, mask=mask)
                return cnt + cs[_L - 1]

            cnt = lax.fori_loop(0, nch, chunk_body, jnp.int32(0))
            ridx = jnp.full((_L,), gbase + rl, jnp.int32)
            plsc.store_scatter(counts_v, [ridx],
                               jnp.broadcast_to(cnt, (_L,)), mask=lanei == 0)
            hwv = jnp.where(lanei == rl, jnp.broadcast_to(cnt, (_L,)), hwv)

        pltpu.async_copy(
            stag_v.at[b], coords_hbm.at[pl.ds(base + gbase, _GSZ)], dma_sem)
        hws = (hwv, hws[1]) if b == 0 else (hws[0], hwv)
      return hws

    hw_init = jnp.full((_L,), _N_SAMPLE, jnp.int32)
    lax.fori_loop(0, _GROUPS // 2, pair_body, (hw_init, hw_init))
    for b in range(2):
        gt = _GROUPS - 2 + b
        pltpu.make_async_copy(
            stag_v.at[b],
            coords_hbm.at[pl.ds(base + gt * _GSZ, _GSZ)],
            dma_sem).wait()
    pltpu.sync_copy(counts_v, counts_hbm.at[pl.ds(base, _RAYS_PER_W)])


@functools.partial(
    pl.kernel,
    mesh=_mesh,
    out_type=jax.ShapeDtypeStruct((_N_RAYS,), jnp.int32),
    scratch_types=[
        pltpu.VMEM((_N_RAYS,), jnp.int32),
        pltpu.VMEM((_N_RAYS,), jnp.int32),
    ],
    compiler_params=pltpu.CompilerParams(needs_layout_passes=False),
)
def _offsets_kernel(counts_hbm, offs_hbm, cv, ov):
    wid = lax.axis_index("s") * _NC + lax.axis_index("c")

    @pl.when(wid == 0)
    def _():
        pltpu.sync_copy(counts_hbm, cv)

        def body(k, b):
            x = cv[pl.ds(k * _L, _L)]
            cs = plsc.cumsum(x)
            ov[pl.ds(k * _L, _L)] = (b + cs) - x
            return b + jnp.sum(x)

        lax.fori_loop(0, _N_RAYS // _L, body, jnp.int32(0))
        pltpu.sync_copy(ov, offs_hbm)


def kernel(rays_o, rays_d, bitfield, n_sample, min_step, max_step, aabb_range, n_grid, n_cascades):
    bits = bitfield.reshape(_NWORDS, 32).astype(jnp.uint32)
    words = jnp.sum(bits << jnp.arange(32, dtype=jnp.uint32)[None, :],
                    axis=1, dtype=jnp.uint32)
    words = lax.bitcast_convert_type(words, jnp.int32)

    rays = jnp.concatenate([rays_o.T, rays_d.T], axis=0)

    ms = jnp.asarray(min_step, jnp.float32)
    params = jnp.concatenate([
        aabb_range[0].astype(jnp.float32),
        aabb_range[1].astype(jnp.float32),
        ms[None],
        jnp.zeros((9,), jnp.float32),
    ])

    coords_flat, counts = _march_kernel(rays, words, params)
    offsets = _offsets_kernel(counts)
    coords = coords_flat.reshape(_N_RAYS, _N_SAMPLE, 7)
    numsteps_out = jnp.stack([counts, offsets], axis=-1)
    return coords, numsteps_out


# speedup vs baseline: 4.0155x; 1.0245x over previous
import functools

import jax
import jax.numpy as jnp
from jax import lax
from jax.experimental import pallas as pl
from jax.experimental.pallas import tpu as pltpu
from jax.experimental.pallas import tpu_sc as plsc

_N_RAYS = 8192
_N_SAMPLE = 256
_N_GRID = 128
_NEAR = 0.2
_ROW = _N_SAMPLE * 7
_GSZ = 16

_info = plsc.get_sparse_core_info()
_NC, _NS, _L = _info.num_cores, _info.num_subcores, _info.num_lanes
_NW = _NC * _NS
_RAYS_PER_W = _N_RAYS // _NW
_GROUPS = _RAYS_PER_W // _GSZ
_NWORDS = _N_GRID * _N_GRID * _N_GRID // 32

_mesh = plsc.VectorSubcoreMesh(core_axis_name="c", subcore_axis_name="s")


@functools.partial(
    pl.kernel,
    mesh=_mesh,
    out_type=[
        jax.ShapeDtypeStruct((_N_RAYS, _ROW), jnp.float32),
        jax.ShapeDtypeStruct((_N_RAYS,), jnp.int32),
    ],
    scratch_types=[
        pltpu.VMEM((_NWORDS,), jnp.int32),
        pltpu.VMEM((6, _RAYS_PER_W), jnp.float32),
        pltpu.VMEM((16,), jnp.float32),
        pltpu.VMEM((_GSZ, _ROW), jnp.float32),
        pltpu.VMEM((_RAYS_PER_W,), jnp.int32),
    ],
    compiler_params=pltpu.CompilerParams(needs_layout_passes=False),
)
def _march_kernel(rays_hbm, words_hbm, params_hbm,
                  coords_hbm, counts_hbm,
                  bits_v, rays_v, par_v, stag_v, counts_v):
    wid = lax.axis_index("s") * _NC + lax.axis_index("c")
    base = wid * _RAYS_PER_W

    pltpu.sync_copy(words_hbm, bits_v)
    pltpu.sync_copy(rays_hbm.at[:, pl.ds(base, _RAYS_PER_W)], rays_v)
    pltpu.sync_copy(params_hbm, par_v)

    pv = par_v[pl.ds(0, 16)]
    a0x, a0y, a0z = pv[0], pv[1], pv[2]
    a1x, a1y, a1z = pv[3], pv[4], pv[5]
    ms = pv[6]
    sx, sy, sz = a1x - a0x, a1y - a0y, a1z - a0z

    lanef = lax.iota(jnp.int32, _L).astype(jnp.float32)
    lanei = lax.iota(jnp.int32, _L)
    zv = jnp.zeros((_L,), jnp.float32)
    ngf = jnp.float32(_N_GRID)

    dtv = jnp.broadcast_to(ms, (_L,))

    def group_body(g, hwv):
        gbase = g * _GSZ
        oxv = rays_v[0, pl.ds(gbase, _GSZ)]
        oyv = rays_v[1, pl.ds(gbase, _GSZ)]
        ozv = rays_v[2, pl.ds(gbase, _GSZ)]
        dxv = rays_v[3, pl.ds(gbase, _GSZ)]
        dyv = rays_v[4, pl.ds(gbase, _GSZ)]
        dzv = rays_v[5, pl.ds(gbase, _GSZ)]

        sdx = jnp.where(jnp.abs(dxv) > 1e-10, dxv, jnp.float32(1e-10))
        sdy = jnp.where(jnp.abs(dyv) > 1e-10, dyv, jnp.float32(1e-10))
        sdz = jnp.where(jnp.abs(dzv) > 1e-10, dzv, jnp.float32(1e-10))
        ivx, ivy, ivz = 1.0 / sdx, 1.0 / sdy, 1.0 / sdz
        t0x, t1x = (a0x - oxv) * ivx, (a1x - oxv) * ivx
        t0y, t1y = (a0y - oyv) * ivy, (a1y - oyv) * ivy
        t0z, t1z = (a0z - ozv) * ivz, (a1z - ozv) * ivz
        tnv = jnp.maximum(jnp.maximum(jnp.minimum(t0x, t1x),
                                      jnp.minimum(t0y, t1y)),
                          jnp.minimum(t0z, t1z))
        tfv = jnp.minimum(jnp.minimum(jnp.maximum(t0x, t1x),
                                      jnp.maximum(t0y, t1y)),
                          jnp.maximum(t0z, t1z))
        tnv = jnp.maximum(tnv, jnp.float32(_NEAR))
        qv = (tfv - tnv) / ms
        nchv = jnp.minimum(
            jnp.int32(_N_SAMPLE // _L),
            jnp.maximum(jnp.int32(0),
                        (qv * jnp.float32(1.0 / _L)).astype(jnp.int32) + 1))

        for rl in range(_GSZ):
            ox, oy, oz = oxv[rl], oyv[rl], ozv[rl]
            dx, dy, dz = dxv[rl], dyv[rl], dzv[rl]
            tn, tf, nch = tnv[rl], tfv[rl], nchv[rl]

            khi = lax.shift_right_logical(hwv[rl] * 7 + (_L - 1), 4)

            def zero_body(k, _, rl=rl):
                stag_v[rl, pl.ds(k * _L, _L)] = zv
                return 0
            lax.fori_loop(0, khi, zero_body, 0)

            rl_splat = jnp.full((_L,), rl, jnp.int32)
            dxs = jnp.broadcast_to(dx, (_L,))
            dys = jnp.broadcast_to(dy, (_L,))
            dzs = jnp.broadcast_to(dz, (_L,))

            def chunk_body(c, cnt, tn=tn, tf=tf, dx=dx, dy=dy, dz=dz,
                           ox=ox, oy=oy, oz=oz, rl_splat=rl_splat,
                           dtv=dtv, dxs=dxs, dys=dys, dzs=dzs):
                jv = lanef + (c * _L).astype(jnp.float32)
                tv = tn + (jv + 0.5) * ms
                px = ox + tv * dx
                py = oy + tv * dy
                pz = oz + tv * dz
                inb = ((px >= a0x) & (px <= a1x) &
                       (py >= a0y) & (py <= a1y) &
                       (pz >= a0z) & (pz <= a1z))
                seg = tv < tf
                gx = jnp.clip(((px - a0x) / sx * ngf).astype(jnp.int32),
                              0, _N_GRID - 1)
                gy = jnp.clip(((py - a0y) / sy * ngf).astype(jnp.int32),
                              0, _N_GRID - 1)
                gz = jnp.clip(((pz - a0z) / sz * ngf).astype(jnp.int32),
                              0, _N_GRID - 1)
                lin = (gx * _N_GRID + gy) * _N_GRID + gz
                wi = lax.shift_right_logical(lin, 5)
                bi = lin & 31
                w = plsc.load_gather(bits_v, [wi])
                occ = (lax.shift_right_logical(w, bi) & 1) == 1
                mask = inb & seg & occ
                mi = mask.astype(jnp.int32)
                cs = plsc.cumsum(mi)
                pos = (cnt + cs - 1) * 7
                plsc.store_scatter(stag_v, [rl_splat, pos], px, mask=mask)
                plsc.store_scatter(stag_v, [rl_splat, pos + 1], py, mask=mask)
                plsc.store_scatter(stag_v, [rl_splat, pos + 2], pz, mask=mask)
                plsc.store_scatter(stag_v, [rl_splat, pos + 3], dtv, mask=mask)
                plsc.store_scatter(stag_v, [rl_splat, pos + 4], dxs, mask=mask)
                plsc.store_scatter(stag_v, [rl_splat, pos + 5], dys, mask=mask)
                plsc.store_scatter(stag_v, [rl_splat, pos + 6], dzs, mask=mask)
                return cnt + cs[_L - 1]

            cnt = lax.fori_loop(0, nch, chunk_body, jnp.int32(0))
            ridx = jnp.full((_L,), gbase + rl, jnp.int32)
            plsc.store_scatter(counts_v, [ridx],
                               jnp.broadcast_to(cnt, (_L,)), mask=lanei == 0)
            hwv = jnp.where(lanei == rl, jnp.broadcast_to(cnt, (_L,)), hwv)

        pltpu.sync_copy(
            stag_v, coords_hbm.at[pl.ds(base + gbase, _GSZ)])
        return hwv

    lax.fori_loop(0, _GROUPS, group_body,
                  jnp.full((_L,), _N_SAMPLE, jnp.int32))
    pltpu.sync_copy(counts_v, counts_hbm.at[pl.ds(base, _RAYS_PER_W)])


@functools.partial(
    pl.kernel,
    mesh=_mesh,
    out_type=jax.ShapeDtypeStruct((_N_RAYS,), jnp.int32),
    scratch_types=[
        pltpu.VMEM((_N_RAYS,), jnp.int32),
        pltpu.VMEM((_N_RAYS,), jnp.int32),
    ],
    compiler_params=pltpu.CompilerParams(needs_layout_passes=False),
)
def _offsets_kernel(counts_hbm, offs_hbm, cv, ov):
    wid = lax.axis_index("s") * _NC + lax.axis_index("c")

    @pl.when(wid == 0)
    def _():
        pltpu.sync_copy(counts_hbm, cv)

        def body(k, b):
            x = cv[pl.ds(k * _L, _L)]
            cs = plsc.cumsum(x)
            ov[pl.ds(k * _L, _L)] = (b + cs) - x
            return b + jnp.sum(x)

        lax.fori_loop(0, _N_RAYS // _L, body, jnp.int32(0))
        pltpu.sync_copy(ov, offs_hbm)


def kernel(rays_o, rays_d, bitfield, n_sample, min_step, max_step, aabb_range, n_grid, n_cascades):
    bits = bitfield.reshape(_NWORDS, 32).astype(jnp.uint32)
    words = jnp.sum(bits << jnp.arange(32, dtype=jnp.uint32)[None, :],
                    axis=1, dtype=jnp.uint32)
    words = lax.bitcast_convert_type(words, jnp.int32)

    rays = jnp.concatenate([rays_o.T, rays_d.T], axis=0)

    ms = jnp.asarray(min_step, jnp.float32)
    params = jnp.concatenate([
        aabb_range[0].astype(jnp.float32),
        aabb_range[1].astype(jnp.float32),
        ms[None],
        jnp.zeros((9,), jnp.float32),
    ])

    coords_flat, counts = _march_kernel(rays, words, params)
    offsets = _offsets_kernel(counts)
    coords = coords_flat.reshape(_N_RAYS, _N_SAMPLE, 7)
    numsteps_out = jnp.stack([counts, offsets], axis=-1)
    return coords, numsteps_out
